# baseline (speedup 1.0000x reference)
import functools

import jax
import jax.numpy as jnp
from jax.experimental import pallas as pl

N = 10000
E = 160000
F = 256
H3 = 3 * F
_C = 20.0

_f32 = jnp.float32


def _leaky(x):
    return jnp.maximum(x, 0.01 * x)


def _elu(x):
    return jnp.where(x > 0, x, jnp.exp(jnp.minimum(x, 0.0)) - 1.0)


_BR = 400
_GRID_N = N // _BR


def _full(shape):
    return pl.BlockSpec(shape, lambda i: tuple(0 for _ in shape))


def _rows(shape):
    return pl.BlockSpec(shape, lambda i: (i,) + tuple(0 for _ in shape[1:]))


def _pre_body(x_ref, wpn_ref, bpn_ref, wn_ref, w2d_ref, b2c_ref,
              hv_ref, pn1_ref, hvd_ref):
    x = x_ref[...]
    h = _leaky(jnp.dot(x, wpn_ref[...], preferred_element_type=_f32)
               + bpn_ref[...])
    hv_ref[...] = h
    pn1_ref[...] = jnp.dot(x, wn_ref[...], preferred_element_type=_f32)
    hvd_ref[...] = (jnp.dot(h, w2d_ref[...], preferred_element_type=_f32)
                    + b2c_ref[...])


def _tc_pre(x, wpn_t, bpn, wn_t, w2d, b2c):
    return pl.pallas_call(
        _pre_body,
        grid=(_GRID_N,),
        in_specs=[_rows((_BR, F)), _full((F, F)), _full((1, F)),
                  _full((F, F)), _full((F, 1)), _full((1, 1))],
        out_specs=[_rows((_BR, F)), _rows((_BR, F)), _rows((_BR, 1))],
        out_shape=[jax.ShapeDtypeStruct((N, F), _f32),
                   jax.ShapeDtypeStruct((N, F), _f32),
                   jax.ShapeDtypeStruct((N, 1), _f32)],
    )(x, wpn_t, bpn, wn_t, w2d, b2c)


_BE = 1600
_GRID_E = E // _BE


def _eproj_body(ef_ref, we_ref, b_ref, r_ref):
    r_ref[...] = (jnp.dot(ef_ref[...], we_ref[...],
                          preferred_element_type=_f32) + b_ref[...])


def _tc_edgeproj(ef, we_t, b1):
    return pl.pallas_call(
        _eproj_body,
        grid=(_GRID_E,),
        in_specs=[_rows((_BE, 16)), _full((16, F)), _full((1, F))],
        out_specs=_rows((_BE, F)),
        out_shape=jax.ShapeDtypeStruct((E, F), _f32),
    )(ef, we_t, b1)


def _gru_math(x, h, wi_t, wh_t, bi, bh):
    gi = jnp.dot(x, wi_t, preferred_element_type=_f32) + bi
    gh = jnp.dot(h, wh_t, preferred_element_type=_f32) + bh
    r = jax.nn.sigmoid(gi[:, :F] + gh[:, :F])
    z = jax.nn.sigmoid(gi[:, F:2 * F] + gh[:, F:2 * F])
    n = jnp.tanh(gi[:, 2 * F:] + r * gh[:, 2 * F:])
    return (1.0 - z) * n + z * h


def _gru_l0_body(msg_ref, st_ref, etw_ref, etb_ref, wi_ref, wh_ref,
                 bi_ref, bh_ref, h_ref, out_ref):
    ind = jnp.where(st_ref[...] > 0, 1.0, 0.0)
    ctx = _elu(jnp.dot(msg_ref[...], etw_ref[...],
                       preferred_element_type=_f32) + ind * etb_ref[...])
    out_ref[...] = jax.nn.relu(
        _gru_math(ctx, h_ref[...], wi_ref[...], wh_ref[...],
                  bi_ref[...], bh_ref[...]))


def _tc_gru_l0(msg, st, etw_t, etb, wi_t, wh_t, bi, bh, h):
    return pl.pallas_call(
        _gru_l0_body,
        grid=(_GRID_N,),
        in_specs=[_rows((_BR, F)), _rows((_BR, 1)), _full((F, F)),
                  _full((1, F)), _full((F, H3)), _full((F, H3)),
                  _full((1, H3)), _full((1, H3)), _rows((_BR, F))],
        out_specs=_rows((_BR, F)),
        out_shape=jax.ShapeDtypeStruct((N, F), _f32),
    )(msg, st, etw_t, etb, wi_t, wh_t, bi, bh, h)


def _gru_l12_body(msg_ref, wi_ref, wh_ref, bi_ref, bh_ref, h_ref, out_ref):
    ctx = _elu(msg_ref[...])
    out_ref[...] = jax.nn.relu(
        _gru_math(ctx, h_ref[...], wi_ref[...], wh_ref[...],
                  bi_ref[...], bh_ref[...]))


def _tc_gru_l12(msg, wi_t, wh_t, bi, bh, h):
    return pl.pallas_call(
        _gru_l12_body,
        grid=(_GRID_N,),
        in_specs=[_rows((_BR, F)), _full((F, H3)), _full((F, H3)),
                  _full((1, H3)), _full((1, H3)), _rows((_BR, F))],
        out_specs=_rows((_BR, F)),
        out_shape=jax.ShapeDtypeStruct((N, F), _f32),
    )(msg, wi_t, wh_t, bi, bh, h)


def _prep_body(x_ref, wpn_ref, bpn_ref, wd_ref, bdc_ref, ws_ref,
               hvp_ref, sd_ref, ss_ref):
    x = x_ref[...]
    hvp_ref[...] = (jnp.dot(x, wpn_ref[...], preferred_element_type=_f32)
                    + bpn_ref[...])
    sd_ref[...] = (jnp.dot(x, wd_ref[...], preferred_element_type=_f32)
                   + bdc_ref[...])
    ss_ref[...] = jnp.dot(x, ws_ref[...], preferred_element_type=_f32)


def _tc_prep(x, wpn_t, bpn, wd, bdc, ws):
    return pl.pallas_call(
        _prep_body,
        grid=(_GRID_N,),
        in_specs=[_rows((_BR, F)), _full((F, F)), _full((1, F)),
                  _full((F, 1)), _full((1, 1)), _full((F, 1))],
        out_specs=[_rows((_BR, F)), _rows((_BR, 1)), _rows((_BR, 1))],
        out_shape=[jax.ShapeDtypeStruct((N, F), _f32),
                   jax.ShapeDtypeStruct((N, 1), _f32),
                   jax.ShapeDtypeStruct((N, 1), _f32)],
    )(x, wpn_t, bpn, wd, bdc, ws)


def _readout_body(node_ref, wg0_ref, w0_ref, b0_ref, pn0_ref, bpn0_ref,
                  wi0_ref, wh0_ref, bi0_ref, bh0_ref,
                  wg1_ref, w1_ref, b1_ref, pn1_ref, bpn1_ref,
                  wi1_ref, wh1_ref, bi1_ref, bh1_ref, out_ref):
    node = node_ref[...]
    g = jnp.sum(node, axis=0, keepdims=True)

    def step(g, wg, wcl, bcl, pnw, bpn, wi, wh, bi, bh):
        c = jnp.dot(jax.nn.relu(g), wg, preferred_element_type=_f32) + bcl
        z = _leaky(jnp.dot(node, wcl, preferred_element_type=_f32) + c)
        z = z - jnp.max(z)
        a = jnp.exp(z)
        a = a / jnp.sum(a)
        v = jnp.dot(a.T, node, preferred_element_type=_f32)
        g_repr = jnp.dot(v, pnw, preferred_element_type=_f32) + bpn
        ctx = _elu(g_repr)
        return _gru_math(ctx, g, wi, wh, bi, bh)

    g = step(g, wg0_ref[...], w0_ref[...], b0_ref[...], pn0_ref[...],
             bpn0_ref[...], wi0_ref[...], wh0_ref[...], bi0_ref[...],
             bh0_ref[...])
    g = step(g, wg1_ref[...], w1_ref[...], b1_ref[...], pn1_ref[...],
             bpn1_ref[...], wi1_ref[...], wh1_ref[...], bi1_ref[...],
             bh1_ref[...])
    out_ref[...] = g


def _tc_readout(node, args0, args1):
    specs = [_full((N, F))]
    ins = [node]
    for (wg, wcl, bcl, pnw, bpn, wi, wh, bi, bh) in (args0, args1):
        specs += [_full((F, 1)), _full((F, 1)), _full((1, 1)),
                  _full((F, F)), _full((1, F)),
                  _full((F, H3)), _full((F, H3)), _full((1, H3)),
                  _full((1, H3))]
        ins += [wg, wcl, bcl, pnw, bpn, wi, wh, bi, bh]
    return pl.pallas_call(
        _readout_body,
        grid=(1,),
        in_specs=specs,
        out_specs=_full((1, F)),
        out_shape=jax.ShapeDtypeStruct((1, F), _f32),
    )(*ins)


def _sparse_layer0(pn1, r, hvd, w2e, src, dst):
    he1 = _leaky(pn1[src] + r)
    t = he1 @ w2e
    e0 = jnp.exp(_leaky(hvd[dst, 0] + t) - _C)
    s = jax.ops.segment_sum(e0, dst, num_segments=N)
    a = e0 / jnp.maximum(s[dst], 1e-30)
    msg = jax.ops.segment_sum(a[:, None] * he1, dst, num_segments=N)
    return msg, s.reshape(N, 1)


def _sparse_layer12(sd, ss, hvp, src, dst):
    e = jnp.exp(_leaky(sd[dst, 0] + ss[src, 0]) - _C)
    s = jax.ops.segment_sum(e, dst, num_segments=N)
    a = e / jnp.maximum(s[dst], 1e-30)
    return jax.ops.segment_sum(a[:, None] * hvp[src], dst, num_segments=N)


def kernel(node_feats, edge_feats, edge_index, params):
    p = params
    src = edge_index[0]
    dst = edge_index[1]

    def row(v):
        return v.reshape(1, -1)

    def col(v):
        return v.reshape(-1, 1)

    wpe1 = p['gc_pe1_W']
    wn_t = wpe1[:, :F].T
    we_t = wpe1[:, F:].T
    w2 = p['gc_pe2_W'][0]
    b2c = jnp.full((1, 1), p['gc_pe2_b'][0], _f32)

    hv_new, pn1, hvd = _tc_pre(node_feats, p['gc_pn_W'].T, row(p['gc_pn_b']),
                               wn_t, col(w2[:F]), b2c)
    r = _tc_edgeproj(edge_feats, we_t, row(p['gc_pe1_b']))
    msg, st = _sparse_layer0(pn1, r, hvd, w2[F:], src, dst)
    node = _tc_gru_l0(msg, st, p['gc_et_W'].T, row(p['gc_et_b']),
                      p['gc_gru_Wi'].T, p['gc_gru_Wh'].T,
                      row(p['gc_gru_bi']), row(p['gc_gru_bh']), hv_new)

    for i in range(2):
        pre = 'l%d_' % i
        wpe = p[pre + 'pe_W'][0]
        bdc = jnp.full((1, 1), p[pre + 'pe_b'][0], _f32)
        hvp, sd, ss = _tc_prep(node, p[pre + 'pn_W'].T, row(p[pre + 'pn_b']),
                               col(wpe[:F]), bdc, col(wpe[F:]))
        msg = _sparse_layer12(sd, ss, hvp, src, dst)
        node = _tc_gru_l12(msg, p[pre + 'gru_Wi'].T, p[pre + 'gru_Wh'].T,
                           row(p[pre + 'gru_bi']), row(p[pre + 'gru_bh']),
                           node)

    ro = []
    for t in range(2):
        pre = 'r%d_' % t
        ro.append((col(p[pre + 'cl_W'][0, :F]), col(p[pre + 'cl_W'][0, F:]),
                   jnp.full((1, 1), p[pre + 'cl_b'][0], _f32),
                   p[pre + 'pn_W'].T, row(p[pre + 'pn_b']),
                   p[pre + 'gru_Wi'].T, p[pre + 'gru_Wh'].T,
                   row(p[pre + 'gru_bi']), row(p[pre + 'gru_bh'])))
    return _tc_readout(node, ro[0], ro[1])
